# Initial kernel scaffold; baseline (speedup 1.0000x reference)
import jax
import jax.numpy as jnp
from jax.experimental import pallas as pl


def kernel(x, edge_index, emb_table, W1, b1, W2, b2, gamma, beta):
    raise NotImplementedError("write your pallas kernel here")


# speedup vs baseline: 5.5449x; 5.5449x over previous
import functools

import jax
import jax.numpy as jnp
from jax import lax
from jax.experimental import pallas as pl
from jax.experimental.pallas import tpu as pltpu
from jax.experimental.pallas import tpu_sc as plsc

NUM_LAYER = 3
EMB = 256
HALF = 128
N_NODES = 10000
N_EDGES = 160000
EPS = 1e-5

NSUB = 16
CH = 128

CHUNKS = 80
IDX_BATCH = 40
E_PAD = NSUB * CHUNKS * CH

N_PAD = 10240
ROWS_MAIN = 632
ROWS_LAST = N_NODES - (NSUB - 1) * ROWS_MAIN
ACC_ROWS = N_NODES + 8

_PREC = lax.Precision.DEFAULT


def _seg_sum(h_halves, srcs, dsts):
  mesh = plsc.VectorSubcoreMesh(core_axis_name="c", subcore_axis_name="s")

  @functools.partial(
      pl.kernel,
      out_type=jax.ShapeDtypeStruct((N_NODES, EMB), jnp.float32),
      mesh=mesh,
      scratch_types=[
          pltpu.VMEM((IDX_BATCH, CH), jnp.int32),
          pltpu.VMEM((IDX_BATCH, CH), jnp.int32),
          pltpu.VMEM((CH, HALF), jnp.float32),
          pltpu.VMEM((CH, HALF), jnp.float32),
          pltpu.VMEM_SHARED((ACC_ROWS, HALF), jnp.float32),
          pltpu.SemaphoreType.DMA,
          pltpu.SemaphoreType.DMA,
      ],
  )
  def k(h_hbm, srcs_hbm, dsts_hbm, agg_hbm, sidx, didx, buf0, buf1, acc,
        sem0, sem1):
    c = lax.axis_index("c")
    s = lax.axis_index("s")
    hc = h_hbm.at[c]
    r0 = s * ROWS_MAIN

    @pl.when(s < NSUB - 1)
    def _():
      pltpu.sync_copy(hc.at[pl.ds(r0, ROWS_MAIN)],
                      acc.at[pl.ds(r0, ROWS_MAIN)])

    @pl.when(s == NSUB - 1)
    def _():
      pltpu.sync_copy(hc.at[pl.ds((NSUB - 1) * ROWS_MAIN, ROWS_LAST)],
                      acc.at[pl.ds((NSUB - 1) * ROWS_MAIN, ROWS_LAST)])

    plsc.subcore_barrier()

    bufs = (buf0, buf1)
    sems = (sem0, sem1)
    for half in range(CHUNKS // IDX_BATCH):
      pltpu.sync_copy(srcs_hbm.at[s].at[pl.ds(half * IDX_BATCH, IDX_BATCH)],
                      sidx)
      pltpu.sync_copy(dsts_hbm.at[s].at[pl.ds(half * IDX_BATCH, IDX_BATCH)],
                      didx)
      pltpu.async_copy(hc.at[sidx.at[0]], buf0, sem0)
      pltpu.async_copy(hc.at[sidx.at[1]], buf1, sem1)

      @pl.loop(0, IDX_BATCH, step=2)
      def _(kk):
        for b in range(2):
          kb = kk + b
          pltpu.make_async_copy(hc.at[sidx.at[kb]], bufs[b], sems[b]).wait()
          pltpu.sync_copy(bufs[b], acc.at[didx.at[kb]], add=True)
          nxt = kb + 2

          @pl.when(nxt < IDX_BATCH)
          def _():
            pltpu.async_copy(hc.at[sidx.at[nxt]], bufs[b], sems[b])

    plsc.subcore_barrier()

    col0 = c * HALF

    @pl.when(s < NSUB - 1)
    def _():
      pltpu.sync_copy(acc.at[pl.ds(r0, ROWS_MAIN)],
                      agg_hbm.at[pl.ds(r0, ROWS_MAIN), pl.ds(col0, HALF)])

    @pl.when(s == NSUB - 1)
    def _():
      pltpu.sync_copy(
          acc.at[pl.ds((NSUB - 1) * ROWS_MAIN, ROWS_LAST)],
          agg_hbm.at[pl.ds((NSUB - 1) * ROWS_MAIN, ROWS_LAST),
                     pl.ds(col0, HALF)])

  return k(h_halves, srcs, dsts)


def _emb_gather(emb_halves, x2):
  rows_per_sub = N_PAD // NSUB
  nch = rows_per_sub // CH
  mesh = plsc.VectorSubcoreMesh(core_axis_name="c", subcore_axis_name="s")

  @functools.partial(
      pl.kernel,
      out_type=jax.ShapeDtypeStruct((2, N_PAD, HALF), jnp.float32),
      mesh=mesh,
      scratch_types=[
          pltpu.VMEM((nch, CH), jnp.int32),
          pltpu.VMEM((CH, HALF), jnp.float32),
          pltpu.SemaphoreType.DMA,
      ],
  )
  def k(emb_hbm, x_hbm, h0_hbm, xidx, buf, sem):
    c = lax.axis_index("c")
    s = lax.axis_index("s")
    pltpu.sync_copy(x_hbm.at[s], xidx)
    ec = emb_hbm.at[c]

    @pl.loop(0, nch)
    def _(kb):
      pltpu.async_copy(ec.at[xidx.at[kb]], buf, sem).wait()
      pltpu.sync_copy(
          buf, h0_hbm.at[c].at[pl.ds(s * rows_per_sub + kb * CH, CH)])

  return k(emb_halves, x2)


def _mlp(agg, W1l, b1l, W2l, b2l):
  NB = 1000
  nb = N_NODES // NB

  def body(a_ref, w1_ref, b1_ref, w2_ref, b2_ref, h2_ref, st_ref):
    i = pl.program_id(0)
    h1 = jnp.dot(a_ref[...], w1_ref[...], precision=_PREC,
                 preferred_element_type=jnp.float32)
    h1 = jnp.maximum(h1 + b1_ref[0:1, :], 0.0)
    h2 = jnp.dot(h1, w2_ref[...], precision=_PREC,
                 preferred_element_type=jnp.float32)
    h2 = h2 + b2_ref[0:1, :]
    h2_ref[...] = h2

    @pl.when(i == 0)
    def _():
      st_ref[...] = jnp.zeros_like(st_ref)

    st_ref[0:1, :] += jnp.sum(h2, axis=0)[None, :]
    st_ref[1:2, :] += jnp.sum(h2 * h2, axis=0)[None, :]

  h2, st = pl.pallas_call(
      body,
      grid=(nb,),
      in_specs=[
          pl.BlockSpec((NB, EMB), lambda i: (i, 0)),
          pl.BlockSpec((EMB, 2 * EMB), lambda i: (0, 0)),
          pl.BlockSpec((1, 2 * EMB), lambda i: (0, 0)),
          pl.BlockSpec((2 * EMB, EMB), lambda i: (0, 0)),
          pl.BlockSpec((1, EMB), lambda i: (0, 0)),
      ],
      out_specs=[
          pl.BlockSpec((NB, EMB), lambda i: (i, 0)),
          pl.BlockSpec((8, EMB), lambda i: (0, 0)),
      ],
      out_shape=[
          jax.ShapeDtypeStruct((N_NODES, EMB), jnp.float32),
          jax.ShapeDtypeStruct((8, EMB), jnp.float32),
      ],
  )(agg, W1l, b1l, W2l, b2l)
  return h2, st


def _bn(h2, st, gammal, betal, relu, last):
  NB = 1000
  nb = N_NODES // NB

  def body(h2_ref, st_ref, g_ref, b_ref, o_ref):
    mean = st_ref[0:1, :] * (1.0 / N_NODES)
    var = st_ref[1:2, :] * (1.0 / N_NODES) - mean * mean
    inv = lax.rsqrt(var + EPS)
    scale = g_ref[0:1, :] * inv
    shift = b_ref[0:1, :] - mean * scale
    y = h2_ref[...] * scale + shift
    if relu:
      y = jnp.maximum(y, 0.0)
    if last:
      o_ref[...] = y
    else:
      o_ref[0] = y[:, :HALF]
      o_ref[1] = y[:, HALF:]

  if last:
    out_spec = pl.BlockSpec((NB, EMB), lambda i: (i, 0))
    out_shape = jax.ShapeDtypeStruct((N_NODES, EMB), jnp.float32)
  else:
    out_spec = pl.BlockSpec((2, NB, HALF), lambda i: (0, i, 0))
    out_shape = jax.ShapeDtypeStruct((2, N_NODES, HALF), jnp.float32)

  return pl.pallas_call(
      body,
      grid=(nb,),
      in_specs=[
          pl.BlockSpec((NB, EMB), lambda i: (i, 0)),
          pl.BlockSpec((8, EMB), lambda i: (0, 0)),
          pl.BlockSpec((1, EMB), lambda i: (0, 0)),
          pl.BlockSpec((1, EMB), lambda i: (0, 0)),
      ],
      out_specs=out_spec,
      out_shape=out_shape,
  )(h2, st, gammal, betal)


def kernel(x, edge_index, emb_table, W1, b1, W2, b2, gamma, beta):
  x = x.astype(jnp.int32)
  src = edge_index[0].astype(jnp.int32)
  dst = edge_index[1].astype(jnp.int32)

  pad = E_PAD - N_EDGES
  src_p = jnp.concatenate([src, jnp.zeros((pad,), jnp.int32)])
  dst_p = jnp.concatenate([dst, jnp.full((pad,), N_NODES, jnp.int32)])
  srcs = src_p.reshape(NSUB, CHUNKS, CH)
  dsts = dst_p.reshape(NSUB, CHUNKS, CH)

  x_p = jnp.concatenate([x, jnp.zeros((N_PAD - N_NODES,), jnp.int32)])
  x2 = x_p.reshape(NSUB, (N_PAD // NSUB) // CH, CH)

  emb_halves = emb_table.reshape(emb_table.shape[0], 2, HALF).transpose(1, 0, 2)

  b1r = b1.reshape(NUM_LAYER, 1, 2 * EMB)
  b2r = b2.reshape(NUM_LAYER, 1, EMB)
  gr = gamma.reshape(NUM_LAYER, 1, EMB)
  br = beta.reshape(NUM_LAYER, 1, EMB)

  h = _emb_gather(emb_halves, x2)
  for l in range(NUM_LAYER):
    agg = _seg_sum(h, srcs, dsts)
    h2, st = _mlp(agg, W1[l], b1r[l], W2[l], b2r[l])
    last = l == NUM_LAYER - 1
    h = _bn(h2, st, gr[l], br[l], relu=not last, last=last)
  return h


# speedup vs baseline: 5.7836x; 1.0431x over previous
import functools

import jax
import jax.numpy as jnp
from jax import lax
from jax.experimental import pallas as pl
from jax.experimental.pallas import tpu as pltpu
from jax.experimental.pallas import tpu_sc as plsc

NUM_LAYER = 3
EMB = 256
HALF = 128
N_NODES = 10000
N_EDGES = 160000
EPS = 1e-5

NSUB = 16
CH = 64
RING = 4

CHUNKS = 160
IDX_BATCH = 40
E_PAD = NSUB * CHUNKS * CH

N_PAD = 10240
ROWS_MAIN = 632
ROWS_LAST = N_NODES - (NSUB - 1) * ROWS_MAIN
ACC_ROWS = N_NODES + 8

_PREC = lax.Precision.DEFAULT


def _seg_sum(h_halves, srcs, dsts):
  mesh = plsc.VectorSubcoreMesh(core_axis_name="c", subcore_axis_name="s")

  @functools.partial(
      pl.kernel,
      out_type=jax.ShapeDtypeStruct((N_NODES, EMB), jnp.float32),
      mesh=mesh,
      scratch_types=[
          pltpu.VMEM((IDX_BATCH, CH), jnp.int32),
          pltpu.VMEM((IDX_BATCH, CH), jnp.int32),
          [pltpu.VMEM((CH, HALF), jnp.float32)] * RING,
          pltpu.VMEM_SHARED((ACC_ROWS, HALF), jnp.float32),
          [pltpu.SemaphoreType.DMA] * RING,
          [pltpu.SemaphoreType.DMA] * RING,
      ],
  )
  def k(h_hbm, srcs_hbm, dsts_hbm, agg_hbm, sidx, didx, bufs, acc,
        gsems, ssems):
    c = lax.axis_index("c")
    s = lax.axis_index("s")
    hc = h_hbm.at[c]
    r0 = s * ROWS_MAIN

    @pl.when(s < NSUB - 1)
    def _():
      pltpu.sync_copy(hc.at[pl.ds(r0, ROWS_MAIN)],
                      acc.at[pl.ds(r0, ROWS_MAIN)])

    @pl.when(s == NSUB - 1)
    def _():
      pltpu.sync_copy(hc.at[pl.ds((NSUB - 1) * ROWS_MAIN, ROWS_LAST)],
                      acc.at[pl.ds((NSUB - 1) * ROWS_MAIN, ROWS_LAST)])

    plsc.subcore_barrier()

    for half in range(CHUNKS // IDX_BATCH):
      pltpu.sync_copy(srcs_hbm.at[s].at[pl.ds(half * IDX_BATCH, IDX_BATCH)],
                      sidx)
      pltpu.sync_copy(dsts_hbm.at[s].at[pl.ds(half * IDX_BATCH, IDX_BATCH)],
                      didx)
      for b in range(RING):
        pltpu.async_copy(hc.at[sidx.at[b]], bufs[b], gsems[b])

      @pl.loop(0, IDX_BATCH, step=RING)
      def _(kk):
        for b in range(RING):
          kb = kk + b
          pltpu.make_async_copy(hc.at[sidx.at[kb]], bufs[b], gsems[b]).wait()
          pltpu.async_copy(bufs[b], acc.at[didx.at[kb]], ssems[b], add=True)
          nxt = kb + RING

          @pl.when(nxt < IDX_BATCH)
          def _():
            pltpu.make_async_copy(bufs[b], acc.at[didx.at[kb]],
                                  ssems[b]).wait()
            pltpu.async_copy(hc.at[sidx.at[nxt]], bufs[b], gsems[b])

      for b in range(RING):
        pltpu.make_async_copy(bufs[b], acc.at[didx.at[0]], ssems[b]).wait()

    plsc.subcore_barrier()

    col0 = c * HALF

    @pl.when(s < NSUB - 1)
    def _():
      pltpu.sync_copy(acc.at[pl.ds(r0, ROWS_MAIN)],
                      agg_hbm.at[pl.ds(r0, ROWS_MAIN), pl.ds(col0, HALF)])

    @pl.when(s == NSUB - 1)
    def _():
      pltpu.sync_copy(
          acc.at[pl.ds((NSUB - 1) * ROWS_MAIN, ROWS_LAST)],
          agg_hbm.at[pl.ds((NSUB - 1) * ROWS_MAIN, ROWS_LAST),
                     pl.ds(col0, HALF)])

  return k(h_halves, srcs, dsts)


def _emb_gather(emb_halves, x2):
  rows_per_sub = N_PAD // NSUB
  nch = rows_per_sub // CH
  mesh = plsc.VectorSubcoreMesh(core_axis_name="c", subcore_axis_name="s")

  @functools.partial(
      pl.kernel,
      out_type=jax.ShapeDtypeStruct((2, N_PAD, HALF), jnp.float32),
      mesh=mesh,
      scratch_types=[
          pltpu.VMEM((nch, CH), jnp.int32),
          pltpu.VMEM((CH, HALF), jnp.float32),
          pltpu.SemaphoreType.DMA,
      ],
  )
  def k(emb_hbm, x_hbm, h0_hbm, xidx, buf, sem):
    c = lax.axis_index("c")
    s = lax.axis_index("s")
    pltpu.sync_copy(x_hbm.at[s], xidx)
    ec = emb_hbm.at[c]

    @pl.loop(0, nch)
    def _(kb):
      pltpu.async_copy(ec.at[xidx.at[kb]], buf, sem).wait()
      pltpu.sync_copy(
          buf, h0_hbm.at[c].at[pl.ds(s * rows_per_sub + kb * CH, CH)])

  return k(emb_halves, x2)


def _mlp(agg, W1l, b1l, W2l, b2l):
  NB = 1000
  nb = N_NODES // NB

  def body(a_ref, w1_ref, b1_ref, w2_ref, b2_ref, h2_ref, st_ref):
    i = pl.program_id(0)
    h1 = jnp.dot(a_ref[...], w1_ref[...], precision=_PREC,
                 preferred_element_type=jnp.float32)
    h1 = jnp.maximum(h1 + b1_ref[0:1, :], 0.0)
    h2 = jnp.dot(h1, w2_ref[...], precision=_PREC,
                 preferred_element_type=jnp.float32)
    h2 = h2 + b2_ref[0:1, :]
    h2_ref[...] = h2

    @pl.when(i == 0)
    def _():
      st_ref[...] = jnp.zeros_like(st_ref)

    st_ref[0:1, :] += jnp.sum(h2, axis=0)[None, :]
    st_ref[1:2, :] += jnp.sum(h2 * h2, axis=0)[None, :]

  h2, st = pl.pallas_call(
      body,
      grid=(nb,),
      in_specs=[
          pl.BlockSpec((NB, EMB), lambda i: (i, 0)),
          pl.BlockSpec((EMB, 2 * EMB), lambda i: (0, 0)),
          pl.BlockSpec((1, 2 * EMB), lambda i: (0, 0)),
          pl.BlockSpec((2 * EMB, EMB), lambda i: (0, 0)),
          pl.BlockSpec((1, EMB), lambda i: (0, 0)),
      ],
      out_specs=[
          pl.BlockSpec((NB, EMB), lambda i: (i, 0)),
          pl.BlockSpec((8, EMB), lambda i: (0, 0)),
      ],
      out_shape=[
          jax.ShapeDtypeStruct((N_NODES, EMB), jnp.float32),
          jax.ShapeDtypeStruct((8, EMB), jnp.float32),
      ],
  )(agg, W1l, b1l, W2l, b2l)
  return h2, st


def _bn(h2, st, gammal, betal, relu, last):
  NB = 1000
  nb = N_NODES // NB

  def body(h2_ref, st_ref, g_ref, b_ref, o_ref):
    mean = st_ref[0:1, :] * (1.0 / N_NODES)
    var = st_ref[1:2, :] * (1.0 / N_NODES) - mean * mean
    inv = lax.rsqrt(var + EPS)
    scale = g_ref[0:1, :] * inv
    shift = b_ref[0:1, :] - mean * scale
    y = h2_ref[...] * scale + shift
    if relu:
      y = jnp.maximum(y, 0.0)
    if last:
      o_ref[...] = y
    else:
      o_ref[0] = y[:, :HALF]
      o_ref[1] = y[:, HALF:]

  if last:
    out_spec = pl.BlockSpec((NB, EMB), lambda i: (i, 0))
    out_shape = jax.ShapeDtypeStruct((N_NODES, EMB), jnp.float32)
  else:
    out_spec = pl.BlockSpec((2, NB, HALF), lambda i: (0, i, 0))
    out_shape = jax.ShapeDtypeStruct((2, N_NODES, HALF), jnp.float32)

  return pl.pallas_call(
      body,
      grid=(nb,),
      in_specs=[
          pl.BlockSpec((NB, EMB), lambda i: (i, 0)),
          pl.BlockSpec((8, EMB), lambda i: (0, 0)),
          pl.BlockSpec((1, EMB), lambda i: (0, 0)),
          pl.BlockSpec((1, EMB), lambda i: (0, 0)),
      ],
      out_specs=out_spec,
      out_shape=out_shape,
  )(h2, st, gammal, betal)


def kernel(x, edge_index, emb_table, W1, b1, W2, b2, gamma, beta):
  x = x.astype(jnp.int32)
  src = edge_index[0].astype(jnp.int32)
  dst = edge_index[1].astype(jnp.int32)

  pad = E_PAD - N_EDGES
  src_p = jnp.concatenate([src, jnp.zeros((pad,), jnp.int32)])
  dst_p = jnp.concatenate([dst, jnp.full((pad,), N_NODES, jnp.int32)])
  srcs = src_p.reshape(NSUB, CHUNKS, CH)
  dsts = dst_p.reshape(NSUB, CHUNKS, CH)

  x_p = jnp.concatenate([x, jnp.zeros((N_PAD - N_NODES,), jnp.int32)])
  x2 = x_p.reshape(NSUB, (N_PAD // NSUB) // CH, CH)

  emb_halves = emb_table.reshape(emb_table.shape[0], 2, HALF).transpose(1, 0, 2)

  b1r = b1.reshape(NUM_LAYER, 1, 2 * EMB)
  b2r = b2.reshape(NUM_LAYER, 1, EMB)
  gr = gamma.reshape(NUM_LAYER, 1, EMB)
  br = beta.reshape(NUM_LAYER, 1, EMB)

  h = _emb_gather(emb_halves, x2)
  for l in range(NUM_LAYER):
    agg = _seg_sum(h, srcs, dsts)
    h2, st = _mlp(agg, W1[l], b1r[l], W2[l], b2r[l])
    last = l == NUM_LAYER - 1
    h = _bn(h2, st, gr[l], br[l], relu=not last, last=last)
  return h


# speedup vs baseline: 7.4948x; 1.2959x over previous
import functools

import jax
import jax.numpy as jnp
from jax import lax
from jax.experimental import pallas as pl
from jax.experimental.pallas import tpu as pltpu
from jax.experimental.pallas import tpu_sc as plsc

NUM_LAYER = 3
EMB = 256
QCOL = 64
NQ = 4
N_NODES = 10000
N_EDGES = 160000
EPS = 1e-5

NSUB = 16
CH = 128
RING = 2

CHUNKS = 80
IDX_BATCH = 40
E_PAD = NSUB * CHUNKS * CH

ROWS_MAIN = 632
ROWS_LAST = N_NODES - (NSUB - 1) * ROWS_MAIN
ACC_ROWS = N_NODES + 8

_PREC = lax.Precision.DEFAULT


def _seg_sum(h_quarters, srcs, dsts):
  mesh = plsc.VectorSubcoreMesh(core_axis_name="c", subcore_axis_name="s")

  @functools.partial(
      pl.kernel,
      out_type=jax.ShapeDtypeStruct((NQ, N_NODES, QCOL), jnp.float32),
      mesh=mesh,
      compiler_params=pltpu.CompilerParams(use_tc_tiling_on_sc=False),
      scratch_types=[
          pltpu.VMEM((IDX_BATCH, CH), jnp.int32),
          pltpu.VMEM((IDX_BATCH, CH), jnp.int32),
          [pltpu.VMEM((CH, QCOL), jnp.float32)] * RING,
          pltpu.VMEM_SHARED((N_NODES, QCOL), jnp.float32),
          pltpu.VMEM_SHARED((ACC_ROWS, QCOL), jnp.float32),
          [pltpu.SemaphoreType.DMA] * RING,
          [pltpu.SemaphoreType.DMA] * RING,
      ],
  )
  def k(h_hbm, srcs_hbm, dsts_hbm, agg_hbm, sidx, didx, bufs, hs, acc,
        gsems, ssems):
    c = lax.axis_index("c")
    s = lax.axis_index("s")
    r0 = s * ROWS_MAIN

    for p in range(2):
      q = 2 * c + p
      hq = h_hbm.at[q]

      @pl.when(s < NSUB - 1)
      def _():
        pltpu.sync_copy(hq.at[pl.ds(r0, ROWS_MAIN)],
                        hs.at[pl.ds(r0, ROWS_MAIN)])
        pltpu.sync_copy(hq.at[pl.ds(r0, ROWS_MAIN)],
                        acc.at[pl.ds(r0, ROWS_MAIN)])

      @pl.when(s == NSUB - 1)
      def _():
        lo = (NSUB - 1) * ROWS_MAIN
        pltpu.sync_copy(hq.at[pl.ds(lo, ROWS_LAST)],
                        hs.at[pl.ds(lo, ROWS_LAST)])
        pltpu.sync_copy(hq.at[pl.ds(lo, ROWS_LAST)],
                        acc.at[pl.ds(lo, ROWS_LAST)])

      plsc.subcore_barrier()

      for half in range(CHUNKS // IDX_BATCH):
        pltpu.sync_copy(srcs_hbm.at[s].at[pl.ds(half * IDX_BATCH, IDX_BATCH)],
                        sidx)
        pltpu.sync_copy(dsts_hbm.at[s].at[pl.ds(half * IDX_BATCH, IDX_BATCH)],
                        didx)
        for b in range(RING):
          pltpu.async_copy(hs.at[sidx.at[b]], bufs[b], gsems[b])

        @pl.loop(0, IDX_BATCH, step=RING)
        def _(kk):
          for b in range(RING):
            kb = kk + b
            pltpu.make_async_copy(hs.at[sidx.at[kb]], bufs[b],
                                  gsems[b]).wait()
            pltpu.async_copy(bufs[b], acc.at[didx.at[kb]], ssems[b],
                             add=True)
            nxt = kb + RING

            @pl.when(nxt < IDX_BATCH)
            def _():
              pltpu.make_async_copy(bufs[b], acc.at[didx.at[kb]],
                                    ssems[b]).wait()
              pltpu.async_copy(hs.at[sidx.at[nxt]], bufs[b], gsems[b])

        for b in range(RING):
          pltpu.make_async_copy(bufs[b], acc.at[didx.at[0]], ssems[b]).wait()

      plsc.subcore_barrier()

      @pl.when(s < NSUB - 1)
      def _():
        pltpu.sync_copy(acc.at[pl.ds(r0, ROWS_MAIN)],
                        agg_hbm.at[q].at[pl.ds(r0, ROWS_MAIN)])

      @pl.when(s == NSUB - 1)
      def _():
        lo = (NSUB - 1) * ROWS_MAIN
        pltpu.sync_copy(acc.at[pl.ds(lo, ROWS_LAST)],
                        agg_hbm.at[q].at[pl.ds(lo, ROWS_LAST)])

  return k(h_quarters, srcs, dsts)


def _emb_onehot(x3, embp):
  NB = 1000
  nb = N_NODES // NB
  vocab = embp.shape[0]

  def body(x_ref, e_ref, o_ref):
    xv = x_ref[0, 0]
    iota = lax.broadcasted_iota(jnp.int32, (NB, vocab), 1)
    oh = (iota == xv[:, None]).astype(jnp.float32)
    y = jnp.dot(oh, e_ref[...], precision=lax.Precision.HIGHEST,
                preferred_element_type=jnp.float32)
    for q in range(NQ):
      o_ref[q] = y[:, q * QCOL:(q + 1) * QCOL]

  return pl.pallas_call(
      body,
      grid=(nb,),
      in_specs=[
          pl.BlockSpec((1, 1, NB), lambda i: (i, 0, 0)),
          pl.BlockSpec((vocab, EMB), lambda i: (0, 0)),
      ],
      out_specs=pl.BlockSpec((NQ, NB, QCOL), lambda i: (0, i, 0)),
      out_shape=jax.ShapeDtypeStruct((NQ, N_NODES, QCOL), jnp.float32),
  )(x3, embp)


def _mlp(agg, W1l, b1l, W2l, b2l):
  NB = 1000
  nb = N_NODES // NB

  def body(a_ref, w1_ref, b1_ref, w2_ref, b2_ref, h2_ref, st_ref):
    i = pl.program_id(0)
    a = jnp.concatenate([a_ref[0], a_ref[1], a_ref[2], a_ref[3]], axis=1)
    h1 = jnp.dot(a, w1_ref[...], precision=_PREC,
                 preferred_element_type=jnp.float32)
    h1 = jnp.maximum(h1 + b1_ref[0:1, :], 0.0)
    h2 = jnp.dot(h1, w2_ref[...], precision=_PREC,
                 preferred_element_type=jnp.float32)
    h2 = h2 + b2_ref[0:1, :]
    h2_ref[...] = h2

    @pl.when(i == 0)
    def _():
      st_ref[...] = jnp.zeros_like(st_ref)

    st_ref[0:1, :] += jnp.sum(h2, axis=0)[None, :]
    st_ref[1:2, :] += jnp.sum(h2 * h2, axis=0)[None, :]

  h2, st = pl.pallas_call(
      body,
      grid=(nb,),
      in_specs=[
          pl.BlockSpec((NQ, NB, QCOL), lambda i: (0, i, 0)),
          pl.BlockSpec((EMB, 2 * EMB), lambda i: (0, 0)),
          pl.BlockSpec((1, 2 * EMB), lambda i: (0, 0)),
          pl.BlockSpec((2 * EMB, EMB), lambda i: (0, 0)),
          pl.BlockSpec((1, EMB), lambda i: (0, 0)),
      ],
      out_specs=[
          pl.BlockSpec((NB, EMB), lambda i: (i, 0)),
          pl.BlockSpec((8, EMB), lambda i: (0, 0)),
      ],
      out_shape=[
          jax.ShapeDtypeStruct((N_NODES, EMB), jnp.float32),
          jax.ShapeDtypeStruct((8, EMB), jnp.float32),
      ],
  )(agg, W1l, b1l, W2l, b2l)
  return h2, st


def _bn(h2, st, gammal, betal, relu, last):
  NB = 1000
  nb = N_NODES // NB

  def body(h2_ref, st_ref, g_ref, b_ref, o_ref):
    mean = st_ref[0:1, :] * (1.0 / N_NODES)
    var = st_ref[1:2, :] * (1.0 / N_NODES) - mean * mean
    inv = lax.rsqrt(var + EPS)
    scale = g_ref[0:1, :] * inv
    shift = b_ref[0:1, :] - mean * scale
    y = h2_ref[...] * scale + shift
    if relu:
      y = jnp.maximum(y, 0.0)
    if last:
      o_ref[...] = y
    else:
      for q in range(NQ):
        o_ref[q] = y[:, q * QCOL:(q + 1) * QCOL]

  if last:
    out_spec = pl.BlockSpec((NB, EMB), lambda i: (i, 0))
    out_shape = jax.ShapeDtypeStruct((N_NODES, EMB), jnp.float32)
  else:
    out_spec = pl.BlockSpec((NQ, NB, QCOL), lambda i: (0, i, 0))
    out_shape = jax.ShapeDtypeStruct((NQ, N_NODES, QCOL), jnp.float32)

  return pl.pallas_call(
      body,
      grid=(nb,),
      in_specs=[
          pl.BlockSpec((NB, EMB), lambda i: (i, 0)),
          pl.BlockSpec((8, EMB), lambda i: (0, 0)),
          pl.BlockSpec((1, EMB), lambda i: (0, 0)),
          pl.BlockSpec((1, EMB), lambda i: (0, 0)),
      ],
      out_specs=out_spec,
      out_shape=out_shape,
  )(h2, st, gammal, betal)


def kernel(x, edge_index, emb_table, W1, b1, W2, b2, gamma, beta):
  x = x.astype(jnp.int32)
  src = edge_index[0].astype(jnp.int32)
  dst = edge_index[1].astype(jnp.int32)

  pad = E_PAD - N_EDGES
  pad_src = (jnp.arange(pad, dtype=jnp.int32) * 97) % N_NODES
  src_p = jnp.concatenate([src, pad_src])
  dst_p = jnp.concatenate([dst, jnp.full((pad,), N_NODES, jnp.int32)])
  srcs = src_p.reshape(NSUB, CHUNKS, CH)
  dsts = dst_p.reshape(NSUB, CHUNKS, CH)

  x3 = x.reshape(N_NODES // 1000, 1, 1000)
  embp = jnp.concatenate(
      [emb_table, jnp.zeros((1024 - emb_table.shape[0], EMB), jnp.float32)])

  b1r = b1.reshape(NUM_LAYER, 1, 2 * EMB)
  b2r = b2.reshape(NUM_LAYER, 1, EMB)
  gr = gamma.reshape(NUM_LAYER, 1, EMB)
  br = beta.reshape(NUM_LAYER, 1, EMB)

  h = _emb_onehot(x3, embp)
  for l in range(NUM_LAYER):
    agg = _seg_sum(h, srcs, dsts)
    h2, st = _mlp(agg, W1[l], b1r[l], W2[l], b2r[l])
    last = l == NUM_LAYER - 1
    h = _bn(h2, st, gr[l], br[l], relu=not last, last=last)
  return h


# speedup vs baseline: 7.5764x; 1.0109x over previous
import functools

import jax
import jax.numpy as jnp
from jax import lax
from jax.experimental import pallas as pl
from jax.experimental.pallas import tpu as pltpu
from jax.experimental.pallas import tpu_sc as plsc

NUM_LAYER = 3
EMB = 256
QCOL = 64
NQ = 4
N_NODES = 10000
N_EDGES = 160000
EPS = 1e-5

NSUB = 16
CH = 128
RING = 4

CHUNKS = 80
IDX_BATCH = 40
E_PAD = NSUB * CHUNKS * CH

ROWS_MAIN = 632
ROWS_LAST = N_NODES - (NSUB - 1) * ROWS_MAIN
ACC_ROWS = N_NODES + 8

_PREC = lax.Precision.DEFAULT


def _seg_sum(h_quarters, srcs, dsts):
  mesh = plsc.VectorSubcoreMesh(core_axis_name="c", subcore_axis_name="s")

  @functools.partial(
      pl.kernel,
      out_type=jax.ShapeDtypeStruct((NQ, N_NODES, QCOL), jnp.float32),
      mesh=mesh,
      compiler_params=pltpu.CompilerParams(use_tc_tiling_on_sc=False),
      scratch_types=[
          pltpu.VMEM((IDX_BATCH, CH), jnp.int32),
          pltpu.VMEM((IDX_BATCH, CH), jnp.int32),
          [pltpu.VMEM((CH, QCOL), jnp.float32)] * RING,
          pltpu.VMEM_SHARED((N_NODES, QCOL), jnp.float32),
          pltpu.VMEM_SHARED((ACC_ROWS, QCOL), jnp.float32),
          [pltpu.SemaphoreType.DMA] * RING,
          [pltpu.SemaphoreType.DMA] * RING,
      ],
  )
  def k(h_hbm, srcs_hbm, dsts_hbm, agg_hbm, sidx, didx, bufs, hs, acc,
        gsems, ssems):
    c = lax.axis_index("c")
    s = lax.axis_index("s")
    r0 = s * ROWS_MAIN

    for p in range(2):
      q = 2 * c + p
      hq = h_hbm.at[q]

      @pl.when(s < NSUB - 1)
      def _():
        pltpu.sync_copy(hq.at[pl.ds(r0, ROWS_MAIN)],
                        hs.at[pl.ds(r0, ROWS_MAIN)])
        pltpu.sync_copy(hq.at[pl.ds(r0, ROWS_MAIN)],
                        acc.at[pl.ds(r0, ROWS_MAIN)])

      @pl.when(s == NSUB - 1)
      def _():
        lo = (NSUB - 1) * ROWS_MAIN
        pltpu.sync_copy(hq.at[pl.ds(lo, ROWS_LAST)],
                        hs.at[pl.ds(lo, ROWS_LAST)])
        pltpu.sync_copy(hq.at[pl.ds(lo, ROWS_LAST)],
                        acc.at[pl.ds(lo, ROWS_LAST)])

      plsc.subcore_barrier()

      for half in range(CHUNKS // IDX_BATCH):
        pltpu.sync_copy(srcs_hbm.at[s].at[pl.ds(half * IDX_BATCH, IDX_BATCH)],
                        sidx)
        pltpu.sync_copy(dsts_hbm.at[s].at[pl.ds(half * IDX_BATCH, IDX_BATCH)],
                        didx)
        for b in range(RING):
          pltpu.async_copy(hs.at[sidx.at[b]], bufs[b], gsems[b])

        @pl.loop(0, IDX_BATCH, step=RING)
        def _(kk):
          for b in range(RING):
            kb = kk + b
            pltpu.make_async_copy(hs.at[sidx.at[kb]], bufs[b],
                                  gsems[b]).wait()
            pltpu.async_copy(bufs[b], acc.at[didx.at[kb]], ssems[b],
                             add=True)
            nxt = kb + RING

            @pl.when(nxt < IDX_BATCH)
            def _():
              pltpu.make_async_copy(bufs[b], acc.at[didx.at[kb]],
                                    ssems[b]).wait()
              pltpu.async_copy(hs.at[sidx.at[nxt]], bufs[b], gsems[b])

        for b in range(RING):
          pltpu.make_async_copy(bufs[b], acc.at[didx.at[0]], ssems[b]).wait()

      plsc.subcore_barrier()

      @pl.when(s < NSUB - 1)
      def _():
        pltpu.sync_copy(acc.at[pl.ds(r0, ROWS_MAIN)],
                        agg_hbm.at[q].at[pl.ds(r0, ROWS_MAIN)])

      @pl.when(s == NSUB - 1)
      def _():
        lo = (NSUB - 1) * ROWS_MAIN
        pltpu.sync_copy(acc.at[pl.ds(lo, ROWS_LAST)],
                        agg_hbm.at[q].at[pl.ds(lo, ROWS_LAST)])

  return k(h_quarters, srcs, dsts)


def _emb_onehot(x3, embp):
  NB = 1000
  nb = N_NODES // NB
  vocab = embp.shape[0]

  def body(x_ref, e_ref, o_ref):
    xv = x_ref[0, 0]
    iota = lax.broadcasted_iota(jnp.int32, (NB, vocab), 1)
    oh = (iota == xv[:, None]).astype(jnp.float32)
    y = jnp.dot(oh, e_ref[...], precision=lax.Precision.HIGHEST,
                preferred_element_type=jnp.float32)
    for q in range(NQ):
      o_ref[q] = y[:, q * QCOL:(q + 1) * QCOL]

  return pl.pallas_call(
      body,
      grid=(nb,),
      in_specs=[
          pl.BlockSpec((1, 1, NB), lambda i: (i, 0, 0)),
          pl.BlockSpec((vocab, EMB), lambda i: (0, 0)),
      ],
      out_specs=pl.BlockSpec((NQ, NB, QCOL), lambda i: (0, i, 0)),
      out_shape=jax.ShapeDtypeStruct((NQ, N_NODES, QCOL), jnp.float32),
  )(x3, embp)


def _mlp(agg, W1l, b1l, W2l, b2l):
  NB = 1000
  nb = N_NODES // NB

  def body(a_ref, w1_ref, b1_ref, w2_ref, b2_ref, h2_ref, st_ref):
    i = pl.program_id(0)
    a = jnp.concatenate([a_ref[0], a_ref[1], a_ref[2], a_ref[3]], axis=1)
    h1 = jnp.dot(a, w1_ref[...], precision=_PREC,
                 preferred_element_type=jnp.float32)
    h1 = jnp.maximum(h1 + b1_ref[0:1, :], 0.0)
    h2 = jnp.dot(h1, w2_ref[...], precision=_PREC,
                 preferred_element_type=jnp.float32)
    h2 = h2 + b2_ref[0:1, :]
    h2_ref[...] = h2

    @pl.when(i == 0)
    def _():
      st_ref[...] = jnp.zeros_like(st_ref)

    st_ref[0:1, :] += jnp.sum(h2, axis=0)[None, :]
    st_ref[1:2, :] += jnp.sum(h2 * h2, axis=0)[None, :]

  h2, st = pl.pallas_call(
      body,
      grid=(nb,),
      in_specs=[
          pl.BlockSpec((NQ, NB, QCOL), lambda i: (0, i, 0)),
          pl.BlockSpec((EMB, 2 * EMB), lambda i: (0, 0)),
          pl.BlockSpec((1, 2 * EMB), lambda i: (0, 0)),
          pl.BlockSpec((2 * EMB, EMB), lambda i: (0, 0)),
          pl.BlockSpec((1, EMB), lambda i: (0, 0)),
      ],
      out_specs=[
          pl.BlockSpec((NB, EMB), lambda i: (i, 0)),
          pl.BlockSpec((8, EMB), lambda i: (0, 0)),
      ],
      out_shape=[
          jax.ShapeDtypeStruct((N_NODES, EMB), jnp.float32),
          jax.ShapeDtypeStruct((8, EMB), jnp.float32),
      ],
  )(agg, W1l, b1l, W2l, b2l)
  return h2, st


def _bn(h2, st, gammal, betal, relu, last):
  NB = 1000
  nb = N_NODES // NB

  def body(h2_ref, st_ref, g_ref, b_ref, o_ref):
    mean = st_ref[0:1, :] * (1.0 / N_NODES)
    var = st_ref[1:2, :] * (1.0 / N_NODES) - mean * mean
    inv = lax.rsqrt(var + EPS)
    scale = g_ref[0:1, :] * inv
    shift = b_ref[0:1, :] - mean * scale
    y = h2_ref[...] * scale + shift
    if relu:
      y = jnp.maximum(y, 0.0)
    if last:
      o_ref[...] = y
    else:
      for q in range(NQ):
        o_ref[q] = y[:, q * QCOL:(q + 1) * QCOL]

  if last:
    out_spec = pl.BlockSpec((NB, EMB), lambda i: (i, 0))
    out_shape = jax.ShapeDtypeStruct((N_NODES, EMB), jnp.float32)
  else:
    out_spec = pl.BlockSpec((NQ, NB, QCOL), lambda i: (0, i, 0))
    out_shape = jax.ShapeDtypeStruct((NQ, N_NODES, QCOL), jnp.float32)

  return pl.pallas_call(
      body,
      grid=(nb,),
      in_specs=[
          pl.BlockSpec((NB, EMB), lambda i: (i, 0)),
          pl.BlockSpec((8, EMB), lambda i: (0, 0)),
          pl.BlockSpec((1, EMB), lambda i: (0, 0)),
          pl.BlockSpec((1, EMB), lambda i: (0, 0)),
      ],
      out_specs=out_spec,
      out_shape=out_shape,
  )(h2, st, gammal, betal)


def kernel(x, edge_index, emb_table, W1, b1, W2, b2, gamma, beta):
  x = x.astype(jnp.int32)
  src = edge_index[0].astype(jnp.int32)
  dst = edge_index[1].astype(jnp.int32)

  pad = E_PAD - N_EDGES
  pad_src = (jnp.arange(pad, dtype=jnp.int32) * 97) % N_NODES
  src_p = jnp.concatenate([src, pad_src])
  dst_p = jnp.concatenate([dst, jnp.full((pad,), N_NODES, jnp.int32)])
  srcs = src_p.reshape(NSUB, CHUNKS, CH)
  dsts = dst_p.reshape(NSUB, CHUNKS, CH)

  x3 = x.reshape(N_NODES // 1000, 1, 1000)
  embp = jnp.concatenate(
      [emb_table, jnp.zeros((1024 - emb_table.shape[0], EMB), jnp.float32)])

  b1r = b1.reshape(NUM_LAYER, 1, 2 * EMB)
  b2r = b2.reshape(NUM_LAYER, 1, EMB)
  gr = gamma.reshape(NUM_LAYER, 1, EMB)
  br = beta.reshape(NUM_LAYER, 1, EMB)

  h = _emb_onehot(x3, embp)
  for l in range(NUM_LAYER):
    agg = _seg_sum(h, srcs, dsts)
    h2, st = _mlp(agg, W1[l], b1r[l], W2[l], b2r[l])
    last = l == NUM_LAYER - 1
    h = _bn(h2, st, gr[l], br[l], relu=not last, last=last)
  return h


# speedup vs baseline: 7.6033x; 1.0035x over previous
import functools

import jax
import jax.numpy as jnp
from jax import lax
from jax.experimental import pallas as pl
from jax.experimental.pallas import tpu as pltpu
from jax.experimental.pallas import tpu_sc as plsc

NUM_LAYER = 3
EMB = 256
QCOL = 64
NQ = 4
N_NODES = 10000
N_EDGES = 160000
EPS = 1e-5

NSUB = 16
CH = 128
RING = 4

CHUNKS = 80
IDX_BATCH = 40
E_PAD = NSUB * CHUNKS * CH

ROWS_MAIN = 632
ROWS_LAST = N_NODES - (NSUB - 1) * ROWS_MAIN
ACC_ROWS = N_NODES + 8

_PREC = lax.Precision.DEFAULT


def _seg_sum(h_quarters, srcs, dsts):
  mesh = plsc.VectorSubcoreMesh(core_axis_name="c", subcore_axis_name="s")

  @functools.partial(
      pl.kernel,
      out_type=jax.ShapeDtypeStruct((NQ, N_NODES, QCOL), jnp.float32),
      mesh=mesh,
      compiler_params=pltpu.CompilerParams(use_tc_tiling_on_sc=False),
      scratch_types=[
          pltpu.VMEM((IDX_BATCH, CH), jnp.int32),
          pltpu.VMEM((IDX_BATCH, CH), jnp.int32),
          [pltpu.VMEM((CH, QCOL), jnp.float32)] * RING,
          pltpu.VMEM_SHARED((N_NODES, QCOL), jnp.float32),
          pltpu.VMEM_SHARED((ACC_ROWS, QCOL), jnp.float32),
          [pltpu.SemaphoreType.DMA] * RING,
          [pltpu.SemaphoreType.DMA] * RING,
      ],
  )
  def k(h_hbm, srcs_hbm, dsts_hbm, agg_hbm, sidx, didx, bufs, hs, acc,
        gsems, ssems):
    c = lax.axis_index("c")
    s = lax.axis_index("s")
    r0 = s * ROWS_MAIN

    for p in range(2):
      q = 2 * c + p
      hq = h_hbm.at[q]

      @pl.when(s < NSUB - 1)
      def _():
        pltpu.sync_copy(hq.at[pl.ds(r0, ROWS_MAIN)],
                        hs.at[pl.ds(r0, ROWS_MAIN)])
        pltpu.sync_copy(hq.at[pl.ds(r0, ROWS_MAIN)],
                        acc.at[pl.ds(r0, ROWS_MAIN)])

      @pl.when(s == NSUB - 1)
      def _():
        lo = (NSUB - 1) * ROWS_MAIN
        pltpu.sync_copy(hq.at[pl.ds(lo, ROWS_LAST)],
                        hs.at[pl.ds(lo, ROWS_LAST)])
        pltpu.sync_copy(hq.at[pl.ds(lo, ROWS_LAST)],
                        acc.at[pl.ds(lo, ROWS_LAST)])

      plsc.subcore_barrier()

      for half in range(CHUNKS // IDX_BATCH):
        pltpu.sync_copy(srcs_hbm.at[s].at[pl.ds(half * IDX_BATCH, IDX_BATCH)],
                        sidx)
        pltpu.sync_copy(dsts_hbm.at[s].at[pl.ds(half * IDX_BATCH, IDX_BATCH)],
                        didx)
        for b in range(RING):
          pltpu.async_copy(hs.at[sidx.at[b]], bufs[b], gsems[b])

        @pl.loop(0, IDX_BATCH, step=RING)
        def _(kk):
          for b in range(RING):
            kb = kk + b
            pltpu.make_async_copy(hs.at[sidx.at[kb]], bufs[b],
                                  gsems[b]).wait()
            pltpu.async_copy(bufs[b], acc.at[didx.at[kb]], ssems[b],
                             add=True)
            nxt = kb + RING

            @pl.when(nxt < IDX_BATCH)
            def _():
              pltpu.make_async_copy(bufs[b], acc.at[didx.at[kb]],
                                    ssems[b]).wait()
              pltpu.async_copy(hs.at[sidx.at[nxt]], bufs[b], gsems[b])

        for b in range(RING):
          pltpu.make_async_copy(bufs[b], acc.at[didx.at[0]], ssems[b]).wait()

      plsc.subcore_barrier()

      @pl.when(s < NSUB - 1)
      def _():
        pltpu.sync_copy(acc.at[pl.ds(r0, ROWS_MAIN)],
                        agg_hbm.at[q].at[pl.ds(r0, ROWS_MAIN)])

      @pl.when(s == NSUB - 1)
      def _():
        lo = (NSUB - 1) * ROWS_MAIN
        pltpu.sync_copy(acc.at[pl.ds(lo, ROWS_LAST)],
                        agg_hbm.at[q].at[pl.ds(lo, ROWS_LAST)])

  return k(h_quarters, srcs, dsts)


def _emb_onehot(x3, embp):
  NB = 1000
  nb = N_NODES // NB
  vocab = embp.shape[0]

  def body(x_ref, e_ref, o_ref):
    xv = x_ref[0, 0]
    iota = lax.broadcasted_iota(jnp.int32, (NB, vocab), 1)
    oh = (iota == xv[:, None]).astype(jnp.float32)
    y = jnp.dot(oh, e_ref[...], precision=lax.Precision.HIGHEST,
                preferred_element_type=jnp.float32)
    for q in range(NQ):
      o_ref[q] = y[:, q * QCOL:(q + 1) * QCOL]

  return pl.pallas_call(
      body,
      grid=(nb,),
      in_specs=[
          pl.BlockSpec((1, 1, NB), lambda i: (i, 0, 0)),
          pl.BlockSpec((vocab, EMB), lambda i: (0, 0)),
      ],
      out_specs=pl.BlockSpec((NQ, NB, QCOL), lambda i: (0, i, 0)),
      out_shape=jax.ShapeDtypeStruct((NQ, N_NODES, QCOL), jnp.float32),
  )(x3, embp)


def _mlp_bn(agg, W1l, b1l, W2l, b2l, gammal, betal, relu, last):
  NB = 1000
  nb = N_NODES // NB

  def body(a_ref, w1_ref, b1_ref, w2_ref, b2_ref, g_ref, be_ref, o_ref,
           h2_buf, st_ref):
    i = pl.program_id(0)

    @pl.when(i < nb)
    def _():
      a = jnp.concatenate([a_ref[0], a_ref[1], a_ref[2], a_ref[3]], axis=1)
      h1 = jnp.dot(a, w1_ref[...], precision=_PREC,
                   preferred_element_type=jnp.float32)
      h1 = jnp.maximum(h1 + b1_ref[0:1, :], 0.0)
      h2 = jnp.dot(h1, w2_ref[...], precision=_PREC,
                   preferred_element_type=jnp.float32)
      h2 = h2 + b2_ref[0:1, :]
      h2_buf[pl.ds(i * NB, NB), :] = h2

      @pl.when(i == 0)
      def _():
        st_ref[...] = jnp.zeros_like(st_ref)

      st_ref[0:1, :] += jnp.sum(h2, axis=0)[None, :]
      st_ref[1:2, :] += jnp.sum(h2 * h2, axis=0)[None, :]

    @pl.when(i >= nb)
    def _():
      j = i - nb
      mean = st_ref[0:1, :] * (1.0 / N_NODES)
      var = st_ref[1:2, :] * (1.0 / N_NODES) - mean * mean
      inv = lax.rsqrt(var + EPS)
      scale = g_ref[0:1, :] * inv
      shift = be_ref[0:1, :] - mean * scale
      y = h2_buf[pl.ds(j * NB, NB), :] * scale + shift
      if relu:
        y = jnp.maximum(y, 0.0)
      if last:
        o_ref[...] = y
      else:
        for q in range(NQ):
          o_ref[q] = y[:, q * QCOL:(q + 1) * QCOL]

  def amap(i):
    return (0, jnp.where(i < nb, i, 2 * nb - 1 - i), 0)

  if last:
    out_spec = pl.BlockSpec((NB, EMB),
                            lambda i: (jnp.maximum(i - nb, 0), 0))
    out_shape = jax.ShapeDtypeStruct((N_NODES, EMB), jnp.float32)
  else:
    out_spec = pl.BlockSpec((NQ, NB, QCOL),
                            lambda i: (0, jnp.maximum(i - nb, 0), 0))
    out_shape = jax.ShapeDtypeStruct((NQ, N_NODES, QCOL), jnp.float32)

  return pl.pallas_call(
      body,
      grid=(2 * nb,),
      in_specs=[
          pl.BlockSpec((NQ, NB, QCOL), amap),
          pl.BlockSpec((EMB, 2 * EMB), lambda i: (0, 0)),
          pl.BlockSpec((1, 2 * EMB), lambda i: (0, 0)),
          pl.BlockSpec((2 * EMB, EMB), lambda i: (0, 0)),
          pl.BlockSpec((1, EMB), lambda i: (0, 0)),
          pl.BlockSpec((1, EMB), lambda i: (0, 0)),
          pl.BlockSpec((1, EMB), lambda i: (0, 0)),
      ],
      out_specs=out_spec,
      out_shape=out_shape,
      scratch_shapes=[
          pltpu.VMEM((N_NODES, EMB), jnp.float32),
          pltpu.VMEM((8, EMB), jnp.float32),
      ],
  )(agg, W1l, b1l, W2l, b2l, gammal, betal)


def kernel(x, edge_index, emb_table, W1, b1, W2, b2, gamma, beta):
  x = x.astype(jnp.int32)
  src = edge_index[0].astype(jnp.int32)
  dst = edge_index[1].astype(jnp.int32)

  pad = E_PAD - N_EDGES
  pad_src = (jnp.arange(pad, dtype=jnp.int32) * 97) % N_NODES
  src_p = jnp.concatenate([src, pad_src])
  dst_p = jnp.concatenate([dst, jnp.full((pad,), N_NODES, jnp.int32)])
  srcs = src_p.reshape(NSUB, CHUNKS, CH)
  dsts = dst_p.reshape(NSUB, CHUNKS, CH)

  x3 = x.reshape(N_NODES // 1000, 1, 1000)
  embp = jnp.concatenate(
      [emb_table, jnp.zeros((1024 - emb_table.shape[0], EMB), jnp.float32)])

  b1r = b1.reshape(NUM_LAYER, 1, 2 * EMB)
  b2r = b2.reshape(NUM_LAYER, 1, EMB)
  gr = gamma.reshape(NUM_LAYER, 1, EMB)
  br = beta.reshape(NUM_LAYER, 1, EMB)

  h = _emb_onehot(x3, embp)
  for l in range(NUM_LAYER):
    agg = _seg_sum(h, srcs, dsts)
    last = l == NUM_LAYER - 1
    h = _mlp_bn(agg, W1[l], b1r[l], W2[l], b2r[l], gr[l], br[l],
                relu=not last, last=last)
  return h


# speedup vs baseline: 8.2266x; 1.0820x over previous
import functools

import jax
import jax.numpy as jnp
from jax import lax
from jax.experimental import pallas as pl
from jax.experimental.pallas import tpu as pltpu
from jax.experimental.pallas import tpu_sc as plsc

NUM_LAYER = 3
EMB = 256
QCOL = 64
NQ = 4
N_NODES = 10000
N_EDGES = 160000
EPS = 1e-5

NSUB = 16
CH = 128
RING = 4

CHUNKS = 80
IDX_BATCH = 40
E_PAD = NSUB * CHUNKS * CH

ROWS_MAIN = 632
ROWS_LAST = N_NODES - (NSUB - 1) * ROWS_MAIN
ACC_ROWS = N_NODES + 8

_PREC = lax.Precision.DEFAULT


def _seg_sum(h_quarters, srcs, dsts):
  mesh = plsc.VectorSubcoreMesh(core_axis_name="c", subcore_axis_name="s")

  @functools.partial(
      pl.kernel,
      out_type=jax.ShapeDtypeStruct((NQ, N_NODES, QCOL), jnp.float32),
      mesh=mesh,
      compiler_params=pltpu.CompilerParams(use_tc_tiling_on_sc=False),
      scratch_types=[
          pltpu.VMEM((IDX_BATCH, CH), jnp.int32),
          pltpu.VMEM((IDX_BATCH, CH), jnp.int32),
          [pltpu.VMEM((CH, QCOL), jnp.float32)] * RING,
          pltpu.VMEM_SHARED((N_NODES, QCOL), jnp.float32),
          pltpu.VMEM_SHARED((ACC_ROWS, QCOL), jnp.float32),
          [pltpu.SemaphoreType.DMA] * RING,
          [pltpu.SemaphoreType.DMA] * RING,
      ],
  )
  def k(h_hbm, srcs_hbm, dsts_hbm, agg_hbm, sidx, didx, bufs, hs, acc,
        gsems, ssems):
    c = lax.axis_index("c")
    s = lax.axis_index("s")
    r0 = s * ROWS_MAIN

    for p in range(2):
      q = 2 * c + p
      hq = h_hbm.at[q]

      @pl.when(s < NSUB - 1)
      def _():
        pltpu.sync_copy(hq.at[pl.ds(r0, ROWS_MAIN)],
                        hs.at[pl.ds(r0, ROWS_MAIN)])
        pltpu.sync_copy(hq.at[pl.ds(r0, ROWS_MAIN)],
                        acc.at[pl.ds(r0, ROWS_MAIN)])

      @pl.when(s == NSUB - 1)
      def _():
        lo = (NSUB - 1) * ROWS_MAIN
        pltpu.sync_copy(hq.at[pl.ds(lo, ROWS_LAST)],
                        hs.at[pl.ds(lo, ROWS_LAST)])
        pltpu.sync_copy(hq.at[pl.ds(lo, ROWS_LAST)],
                        acc.at[pl.ds(lo, ROWS_LAST)])

      plsc.subcore_barrier()

      for half in range(CHUNKS // IDX_BATCH):
        pltpu.sync_copy(srcs_hbm.at[s].at[pl.ds(half * IDX_BATCH, IDX_BATCH)],
                        sidx)
        pltpu.sync_copy(dsts_hbm.at[s].at[pl.ds(half * IDX_BATCH, IDX_BATCH)],
                        didx)
        for b in range(RING):
          pltpu.async_copy(hs.at[sidx.at[b]], bufs[b], gsems[b])

        @pl.loop(0, IDX_BATCH, step=RING)
        def _(kk):
          for b in range(RING):
            kb = kk + b
            pltpu.make_async_copy(hs.at[sidx.at[kb]], bufs[b],
                                  gsems[b]).wait()
            pltpu.async_copy(bufs[b], acc.at[didx.at[kb]], ssems[b],
                             add=True)
            nxt = kb + RING

            @pl.when(nxt < IDX_BATCH)
            def _():
              pltpu.make_async_copy(bufs[b], acc.at[didx.at[kb]],
                                    ssems[b]).wait()
              pltpu.async_copy(hs.at[sidx.at[nxt]], bufs[b], gsems[b])

        for b in range(RING):
          pltpu.make_async_copy(bufs[b], acc.at[didx.at[0]], ssems[b]).wait()

      plsc.subcore_barrier()

      @pl.when(s < NSUB - 1)
      def _():
        pltpu.sync_copy(acc.at[pl.ds(r0, ROWS_MAIN)],
                        agg_hbm.at[q].at[pl.ds(r0, ROWS_MAIN)])

      @pl.when(s == NSUB - 1)
      def _():
        lo = (NSUB - 1) * ROWS_MAIN
        pltpu.sync_copy(acc.at[pl.ds(lo, ROWS_LAST)],
                        agg_hbm.at[q].at[pl.ds(lo, ROWS_LAST)])

  return k(h_quarters, srcs, dsts)


def _seg_sum0(emb_q, x4, srcs, dsts):
  mesh = plsc.VectorSubcoreMesh(core_axis_name="c", subcore_axis_name="s")

  @functools.partial(
      pl.kernel,
      out_type=jax.ShapeDtypeStruct((NQ, N_NODES, QCOL), jnp.float32),
      mesh=mesh,
      compiler_params=pltpu.CompilerParams(use_tc_tiling_on_sc=False),
      scratch_types=[
          pltpu.VMEM((IDX_BATCH, CH), jnp.int32),
          pltpu.VMEM((IDX_BATCH, CH), jnp.int32),
          [pltpu.VMEM((CH, QCOL), jnp.float32)] * RING,
          pltpu.VMEM((5, 125), jnp.int32),
          pltpu.VMEM_SHARED((N_NODES, QCOL), jnp.float32),
          pltpu.VMEM_SHARED((ACC_ROWS, QCOL), jnp.float32),
          [pltpu.SemaphoreType.DMA] * RING,
          [pltpu.SemaphoreType.DMA] * RING,
      ],
  )
  def k(emb_hbm, x_hbm, srcs_hbm, dsts_hbm, agg_hbm, sidx, didx, bufs, xidx,
        hs, acc, gsems, ssems):
    c = lax.axis_index("c")
    s = lax.axis_index("s")
    r0 = s * ROWS_MAIN
    pltpu.sync_copy(x_hbm.at[s], xidx)

    for p in range(2):
      q = 2 * c + p
      eq = emb_hbm.at[q]

      @pl.loop(0, 5)
      def _(kb):
        base = s * 625 + kb * 125
        pltpu.async_copy(eq.at[xidx.at[kb]], bufs[0].at[pl.ds(0, 125)],
                         gsems[0]).wait()
        pltpu.sync_copy(bufs[0].at[pl.ds(0, 125)], hs.at[pl.ds(base, 125)])
        pltpu.sync_copy(bufs[0].at[pl.ds(0, 125)], acc.at[pl.ds(base, 125)])

      plsc.subcore_barrier()

      for half in range(CHUNKS // IDX_BATCH):
        pltpu.sync_copy(srcs_hbm.at[s].at[pl.ds(half * IDX_BATCH, IDX_BATCH)],
                        sidx)
        pltpu.sync_copy(dsts_hbm.at[s].at[pl.ds(half * IDX_BATCH, IDX_BATCH)],
                        didx)
        for b in range(RING):
          pltpu.async_copy(hs.at[sidx.at[b]], bufs[b], gsems[b])

        @pl.loop(0, IDX_BATCH, step=RING)
        def _(kk):
          for b in range(RING):
            kb = kk + b
            pltpu.make_async_copy(hs.at[sidx.at[kb]], bufs[b],
                                  gsems[b]).wait()
            pltpu.async_copy(bufs[b], acc.at[didx.at[kb]], ssems[b],
                             add=True)
            nxt = kb + RING

            @pl.when(nxt < IDX_BATCH)
            def _():
              pltpu.make_async_copy(bufs[b], acc.at[didx.at[kb]],
                                    ssems[b]).wait()
              pltpu.async_copy(hs.at[sidx.at[nxt]], bufs[b], gsems[b])

        for b in range(RING):
          pltpu.make_async_copy(bufs[b], acc.at[didx.at[0]], ssems[b]).wait()

      plsc.subcore_barrier()

      @pl.when(s < NSUB - 1)
      def _():
        pltpu.sync_copy(acc.at[pl.ds(r0, ROWS_MAIN)],
                        agg_hbm.at[q].at[pl.ds(r0, ROWS_MAIN)])

      @pl.when(s == NSUB - 1)
      def _():
        lo = (NSUB - 1) * ROWS_MAIN
        pltpu.sync_copy(acc.at[pl.ds(lo, ROWS_LAST)],
                        agg_hbm.at[q].at[pl.ds(lo, ROWS_LAST)])

  return k(emb_q, x4, srcs, dsts)


def _emb_onehot(x3, embp):
  NB = 1000
  nb = N_NODES // NB
  vocab = embp.shape[0]

  def body(x_ref, e_ref, o_ref):
    xv = x_ref[0, 0]
    iota = lax.broadcasted_iota(jnp.int32, (NB, vocab), 1)
    oh = (iota == xv[:, None]).astype(jnp.float32)
    y = jnp.dot(oh, e_ref[...], precision=lax.Precision.HIGHEST,
                preferred_element_type=jnp.float32)
    for q in range(NQ):
      o_ref[q] = y[:, q * QCOL:(q + 1) * QCOL]

  return pl.pallas_call(
      body,
      grid=(nb,),
      in_specs=[
          pl.BlockSpec((1, 1, NB), lambda i: (i, 0, 0)),
          pl.BlockSpec((vocab, EMB), lambda i: (0, 0)),
      ],
      out_specs=pl.BlockSpec((NQ, NB, QCOL), lambda i: (0, i, 0)),
      out_shape=jax.ShapeDtypeStruct((NQ, N_NODES, QCOL), jnp.float32),
  )(x3, embp)


def _mlp_bn(agg, W1l, b1l, W2l, b2l, gammal, betal, relu, last):
  NB = 1000
  nb = N_NODES // NB

  def body(a_ref, w1_ref, b1_ref, w2_ref, b2_ref, g_ref, be_ref, o_ref,
           h2_buf, st_ref):
    i = pl.program_id(0)

    @pl.when(i < nb)
    def _():
      a = jnp.concatenate([a_ref[0], a_ref[1], a_ref[2], a_ref[3]], axis=1)
      h1 = jnp.dot(a, w1_ref[...], precision=_PREC,
                   preferred_element_type=jnp.float32)
      h1 = jnp.maximum(h1 + b1_ref[0:1, :], 0.0)
      h2 = jnp.dot(h1, w2_ref[...], precision=_PREC,
                   preferred_element_type=jnp.float32)
      h2 = h2 + b2_ref[0:1, :]
      h2_buf[pl.ds(i * NB, NB), :] = h2

      @pl.when(i == 0)
      def _():
        st_ref[...] = jnp.zeros_like(st_ref)

      st_ref[0:1, :] += jnp.sum(h2, axis=0)[None, :]
      st_ref[1:2, :] += jnp.sum(h2 * h2, axis=0)[None, :]

    @pl.when(i >= nb)
    def _():
      j = i - nb
      mean = st_ref[0:1, :] * (1.0 / N_NODES)
      var = st_ref[1:2, :] * (1.0 / N_NODES) - mean * mean
      inv = lax.rsqrt(var + EPS)
      scale = g_ref[0:1, :] * inv
      shift = be_ref[0:1, :] - mean * scale
      y = h2_buf[pl.ds(j * NB, NB), :] * scale + shift
      if relu:
        y = jnp.maximum(y, 0.0)
      if last:
        o_ref[...] = y
      else:
        for q in range(NQ):
          o_ref[q] = y[:, q * QCOL:(q + 1) * QCOL]

  def amap(i):
    return (0, jnp.where(i < nb, i, 2 * nb - 1 - i), 0)

  if last:
    out_spec = pl.BlockSpec((NB, EMB),
                            lambda i: (jnp.maximum(i - nb, 0), 0))
    out_shape = jax.ShapeDtypeStruct((N_NODES, EMB), jnp.float32)
  else:
    out_spec = pl.BlockSpec((NQ, NB, QCOL),
                            lambda i: (0, jnp.maximum(i - nb, 0), 0))
    out_shape = jax.ShapeDtypeStruct((NQ, N_NODES, QCOL), jnp.float32)

  return pl.pallas_call(
      body,
      grid=(2 * nb,),
      in_specs=[
          pl.BlockSpec((NQ, NB, QCOL), amap),
          pl.BlockSpec((EMB, 2 * EMB), lambda i: (0, 0)),
          pl.BlockSpec((1, 2 * EMB), lambda i: (0, 0)),
          pl.BlockSpec((2 * EMB, EMB), lambda i: (0, 0)),
          pl.BlockSpec((1, EMB), lambda i: (0, 0)),
          pl.BlockSpec((1, EMB), lambda i: (0, 0)),
          pl.BlockSpec((1, EMB), lambda i: (0, 0)),
      ],
      out_specs=out_spec,
      out_shape=out_shape,
      scratch_shapes=[
          pltpu.VMEM((N_NODES, EMB), jnp.float32),
          pltpu.VMEM((8, EMB), jnp.float32),
      ],
  )(agg, W1l, b1l, W2l, b2l, gammal, betal)


def kernel(x, edge_index, emb_table, W1, b1, W2, b2, gamma, beta):
  x = x.astype(jnp.int32)
  src = edge_index[0].astype(jnp.int32)
  dst = edge_index[1].astype(jnp.int32)

  pad = E_PAD - N_EDGES
  pad_src = (jnp.arange(pad, dtype=jnp.int32) * 97) % N_NODES
  src_p = jnp.concatenate([src, pad_src])
  dst_p = jnp.concatenate([dst, jnp.full((pad,), N_NODES, jnp.int32)])
  srcs = src_p.reshape(NSUB, CHUNKS, CH)
  dsts = dst_p.reshape(NSUB, CHUNKS, CH)

  x4 = x.reshape(NSUB, 5, 125)
  emb_q = emb_table.reshape(emb_table.shape[0], NQ, QCOL).transpose(1, 0, 2)

  b1r = b1.reshape(NUM_LAYER, 1, 2 * EMB)
  b2r = b2.reshape(NUM_LAYER, 1, EMB)
  gr = gamma.reshape(NUM_LAYER, 1, EMB)
  br = beta.reshape(NUM_LAYER, 1, EMB)

  h = None
  for l in range(NUM_LAYER):
    if l == 0:
      agg = _seg_sum0(emb_q, x4, srcs, dsts)
    else:
      agg = _seg_sum(h, srcs, dsts)
    last = l == NUM_LAYER - 1
    h = _mlp_bn(agg, W1[l], b1r[l], W2[l], b2r[l], gr[l], br[l],
                relu=not last, last=last)
  return h
